# baseline (speedup 1.0000x reference)
import functools

import jax
import jax.numpy as jnp
from jax import lax
from jax.experimental import pallas as pl
from jax.experimental.pallas import tpu as pltpu
from jax.experimental.pallas import tpu_sc as plsc

N = 10000
E = 160000
D = 128
H = 128

BN = 256
N_PAD = 10240
E_PAD = 163840
SC_CHUNK = 256


def _sc_gather(table, idx, chunk):
    B = idx.shape[0]
    info = plsc.get_sparse_core_info()
    nw = info.num_cores * info.num_subcores
    b_per_w = B // nw
    nchunk = b_per_w // chunk
    assert b_per_w * nw == B and nchunk * chunk == b_per_w and chunk % 8 == 0
    mesh = plsc.VectorSubcoreMesh(core_axis_name="c", subcore_axis_name="s")

    @functools.partial(
        pl.kernel,
        out_type=jax.ShapeDtypeStruct((B, table.shape[1]), jnp.float32),
        mesh=mesh,
        scratch_types=[
            pltpu.VMEM((chunk,), jnp.int32),
            pltpu.VMEM((chunk, table.shape[1]), jnp.float32),
            pltpu.SemaphoreType.DMA,
        ],
    )
    def k(table_hbm, idx_hbm, out_hbm, idx_v, rows_v, sem):
        wid = lax.axis_index("s") * info.num_cores + lax.axis_index("c")
        base = wid * b_per_w

        def body(j, carry):
            off = base + j * chunk
            pltpu.sync_copy(idx_hbm.at[pl.ds(off, chunk)], idx_v)
            pltpu.async_copy(table_hbm.at[idx_v], rows_v, sem).wait()
            pltpu.sync_copy(rows_v, out_hbm.at[pl.ds(off, chunk)])
            return carry

        lax.fori_loop(0, nchunk, body, 0, unroll=False)

    return k(table, idx)


def _tc_lstm(xg, wih_t, whh_t, bias, run_cnt, run_len, nruns):
    def body(nruns_ref, run_cnt_ref, run_len_ref, xg_ref, wih_ref, whh_ref,
             b_ref, h_ref, c_ref, xbuf, sem):
        h_ref[...] = jnp.zeros((N_PAD, H), jnp.float32)
        c_ref[...] = jnp.zeros((N_PAD, H), jnp.float32)
        wih = wih_ref[...]
        whh = whh_ref[...]
        b = b_ref[...]

        def run_body(k, offset):
            cnt = run_cnt_ref[k]
            ln = run_len_ref[k]
            nblk = (cnt + BN - 1) // BN

            def step_body(s, offset):
                def blk_body(blk, carry):
                    row0 = blk * BN
                    cp = pltpu.make_async_copy(
                        xg_ref.at[pl.ds(offset + row0, BN)], xbuf, sem)
                    cp.start()
                    cp.wait()
                    hb = h_ref[pl.ds(row0, BN)]
                    cb = c_ref[pl.ds(row0, BN)]
                    gates = (
                        jnp.dot(xbuf[...], wih,
                                preferred_element_type=jnp.float32)
                        + jnp.dot(hb, whh, preferred_element_type=jnp.float32)
                        + b)
                    gi = jax.nn.sigmoid(gates[:, 0:H])
                    gf = jax.nn.sigmoid(gates[:, H:2 * H])
                    gg = jnp.tanh(gates[:, 2 * H:3 * H])
                    go = jax.nn.sigmoid(gates[:, 3 * H:4 * H])
                    cn = gf * cb + gi * gg
                    hn = go * jnp.tanh(cn)
                    rowid = row0 + lax.broadcasted_iota(jnp.int32, (BN, H), 0)
                    m = rowid < cnt
                    h_ref[pl.ds(row0, BN)] = jnp.where(m, hn, hb)
                    c_ref[pl.ds(row0, BN)] = jnp.where(m, cn, cb)
                    return carry

                lax.fori_loop(0, nblk, blk_body, 0, unroll=False)
                return offset + cnt

            return lax.fori_loop(0, ln, step_body, offset, unroll=False)

        lax.fori_loop(0, nruns_ref[0], run_body, 0, unroll=False)

    return pl.pallas_call(
        body,
        out_shape=jax.ShapeDtypeStruct((N_PAD, H), jnp.float32),
        in_specs=[
            pl.BlockSpec(memory_space=pltpu.SMEM),
            pl.BlockSpec(memory_space=pltpu.SMEM),
            pl.BlockSpec(memory_space=pltpu.SMEM),
            pl.BlockSpec(memory_space=pl.ANY),
            pl.BlockSpec(memory_space=pltpu.VMEM),
            pl.BlockSpec(memory_space=pltpu.VMEM),
            pl.BlockSpec(memory_space=pltpu.VMEM),
        ],
        out_specs=pl.BlockSpec(memory_space=pltpu.VMEM),
        scratch_shapes=[
            pltpu.VMEM((N_PAD, H), jnp.float32),
            pltpu.VMEM((BN, D), jnp.float32),
            pltpu.SemaphoreType.DMA,
        ],
    )(nruns, run_cnt, run_len, xg, wih_t, whh_t, bias)


def _build_schedule(deg, node_order):
    ds = deg[node_order]
    nxt = jnp.concatenate([ds[1:], jnp.zeros((1,), ds.dtype)])
    is_b = (ds > nxt) & (ds > 0)
    nb = jnp.sum(is_b.astype(jnp.int32))
    bidx = jnp.cumsum(is_b.astype(jnp.int32)) - 1
    kk = jnp.where(is_b, nb - 1 - bidx, N + 1)
    i_arr = jnp.arange(N, dtype=jnp.int32)
    run_val = jnp.zeros((N + 1,), jnp.int32).at[kk].set(
        ds.astype(jnp.int32), mode="drop")
    run_cnt = jnp.zeros((N + 1,), jnp.int32).at[kk].set(
        i_arr + 1, mode="drop")
    prev = jnp.concatenate([jnp.zeros((1,), jnp.int32), run_val[:-1]])
    run_len = run_val - prev
    return run_cnt, run_len, nb.reshape(1)


def kernel(x, edge_index, W_ih, W_hh, b_ih, b_hh):
    src = edge_index[0]
    dst = edge_index[1]
    deg = jnp.bincount(dst, length=N).astype(jnp.int32)

    order = jnp.argsort(dst, stable=True)
    src_s = src[order]
    dst_s = dst[order]
    start = (jnp.cumsum(deg) - deg).astype(jnp.int32)
    t_e = jnp.arange(E, dtype=jnp.int32) - start[dst_s]

    node_order = jnp.argsort(-deg, stable=True)
    rank = jnp.zeros((N,), jnp.int32).at[node_order].set(
        jnp.arange(N, dtype=jnp.int32))
    r_e = rank[dst_s]

    perm = jnp.argsort(t_e * N + r_e)
    src_tm = src_s[perm]
    src_tm_pad = jnp.zeros((E_PAD,), jnp.int32).at[:E].set(src_tm)

    run_cnt, run_len, nruns = _build_schedule(deg, node_order)

    xg = _sc_gather(x, src_tm_pad, SC_CHUNK)

    wih_t = W_ih.T
    whh_t = W_hh.T
    bias = (b_ih + b_hh).reshape(1, 4 * H)

    h_ranked = _tc_lstm(xg, wih_t, whh_t, bias, run_cnt, run_len, nruns)

    rank_pad = jnp.zeros((N_PAD,), jnp.int32).at[:N].set(rank)
    out = _sc_gather(h_ranked, rank_pad, 320)
    return out[:N]
